# baseline (speedup 1.0000x reference)
import functools

import jax
import jax.numpy as jnp
from jax import lax
from jax.experimental import pallas as pl
from jax.experimental.pallas import tpu as pltpu
from jax.experimental.pallas import tpu_sc as plsc

RACE_CARD = 1000
ETH_CARD = 1000
D = 32
BATCH = 16384

NUM_CORES = 2
NUM_SUBCORES = 16
LANES = 16
NW = NUM_CORES * NUM_SUBCORES
B_PER_W = BATCH // NW
ICHUNK = 64
N_ICHUNK = B_PER_W // ICHUNK


def _make_kernel():
    mesh = plsc.VectorSubcoreMesh(core_axis_name="c", subcore_axis_name="s")

    @functools.partial(
        pl.kernel,
        mesh=mesh,
        compiler_params=pltpu.CompilerParams(needs_layout_passes=False),
        out_type=jax.ShapeDtypeStruct((3 * D * BATCH,), jnp.float32),
        scratch_types=[
            pltpu.VMEM((D * RACE_CARD,), jnp.float32),
            pltpu.VMEM((D * ETH_CARD,), jnp.float32),
            pltpu.VMEM((B_PER_W,), jnp.int32),
            pltpu.VMEM((B_PER_W,), jnp.int32),
            pltpu.VMEM((B_PER_W,), jnp.int32),
            pltpu.VMEM((B_PER_W,), jnp.int32),
            pltpu.VMEM((1, ICHUNK, 128), jnp.float32),
            pltpu.VMEM((3 * D * B_PER_W,), jnp.float32),
            pltpu.SemaphoreType.DMA,
            pltpu.SemaphoreType.DMA,
            pltpu.SemaphoreType.DMA,
        ],
    )
    def k(xr_hbm, xe_hbm, racef_hbm, ethf_hbm, ipack_hbm, out_hbm,
          race_v, eth_v, r_idx, e_idx, b_idx, s_v, istage, out_v,
          sem_in, sem_g, sem_out):
        wid = lax.axis_index("s") * NUM_CORES + lax.axis_index("c")
        base = wid * B_PER_W

        ct_r = pltpu.async_copy(racef_hbm, race_v, sem_in)
        ct_e = pltpu.async_copy(ethf_hbm, eth_v, sem_in)
        pltpu.sync_copy(xr_hbm.at[pl.ds(base, B_PER_W)], r_idx)
        pltpu.sync_copy(xe_hbm.at[pl.ds(base, B_PER_W)], e_idx)

        def idx_body(j, carry):
            sl = pl.ds(j * LANES, LANES)
            ii = r_idx[sl] * ETH_CARD + e_idx[sl]
            b_idx[sl] = lax.shift_right_logical(ii, 2)
            s_v[sl] = lax.bitwise_and(ii, 3)
            return carry

        lax.fori_loop(0, B_PER_W // LANES, idx_body, 0, unroll=4)

        lanes = lax.iota(jnp.int32, LANES)
        zeros = jnp.zeros((LANES,), jnp.int32)

        def ichunk_body(t, carry):
            sl = pl.ds(t * ICHUNK, ICHUNK)
            pltpu.async_copy(
                ipack_hbm.at[b_idx.at[sl]], istage.at[0], sem_g).wait()

            def blk_body(b, carry2):
                g0 = t * ICHUNK + b * LANES
                rows = lanes + b * LANES
                colb = s_v[pl.ds(g0, LANES)] * D
                for c in range(D):
                    v = plsc.load_gather(istage, [zeros, rows, colb + c])
                    out_v[pl.ds((2 * D + c) * B_PER_W + g0, LANES)] = v
                return carry2

            lax.fori_loop(0, ICHUNK // LANES, blk_body, 0)
            return carry

        lax.fori_loop(0, N_ICHUNK, ichunk_body, 0)

        ct_r.wait()
        ct_e.wait()

        def small_body(j, carry):
            sl = pl.ds(j * LANES, LANES)
            r_vec = r_idx[sl]
            e_vec = e_idx[sl]
            for f in range(D):
                out_v[pl.ds(f * B_PER_W + j * LANES, LANES)] = (
                    plsc.load_gather(race_v, [r_vec * D + f]))
                out_v[pl.ds((D + f) * B_PER_W + j * LANES, LANES)] = (
                    plsc.load_gather(eth_v, [e_vec * D + f]))
            return carry

        lax.fori_loop(0, B_PER_W // LANES, small_body, 0)

        copies = []
        for f in range(3 * D):
            copies.append(pltpu.async_copy(
                out_v.at[pl.ds(f * B_PER_W, B_PER_W)],
                out_hbm.at[pl.ds(f * BATCH + base, B_PER_W)], sem_out))
        for c in copies:
            c.wait()

    return k


_sc_kernel = _make_kernel()


@jax.jit
def kernel(x, race_table, ethnicity_table, interaction_table):
    x = x.astype(jnp.int32)
    xr = x[:, 0]
    xe = x[:, 1]
    racef = race_table.reshape(-1)
    ethf = ethnicity_table.reshape(-1)
    ipack = jnp.concatenate(
        [interaction_table[s::4] for s in range(4)], axis=1)
    out = _sc_kernel(xr, xe, racef, ethf, ipack)
    return out.reshape(3 * D, BATCH).T


# speedup vs baseline: 19.0645x; 19.0645x over previous
import functools

import jax
import jax.numpy as jnp
from jax import lax
from jax.experimental import pallas as pl
from jax.experimental.pallas import tpu as pltpu
from jax.experimental.pallas import tpu_sc as plsc

RACE_CARD = 1000
ETH_CARD = 1000
D = 32
BATCH = 16384
NTAB = RACE_CARD * ETH_CARD

NUM_CORES = 2
NUM_SUBCORES = 16
LANES = 16
NW = NUM_CORES * NUM_SUBCORES
B_PER_W = BATCH // NW

NBLK = NTAB // 128
BLK_PER_W = NBLK // NW
CHUNK_BLKS = 4
CHUNK_COLS = CHUNK_BLKS * 128
TAIL0 = NBLK * 128
NDUMP = 64
SCRATCH_ROWS = BATCH + NDUMP
XCHUNK = 2048
SBATCH = 64
SCHUNK = 64


def _make_scan_kernel():
    mesh = plsc.VectorSubcoreMesh(core_axis_name="c", subcore_axis_name="s")

    @functools.partial(
        pl.kernel,
        mesh=mesh,
        compiler_params=pltpu.CompilerParams(needs_layout_passes=False),
        out_type=jax.ShapeDtypeStruct((SCRATCH_ROWS, 128), jnp.float32),
        scratch_types=[
            pltpu.VMEM((XCHUNK,), jnp.int32),
            pltpu.VMEM((XCHUNK,), jnp.int32),
            pltpu.VMEM((BATCH,), jnp.int32),
            pltpu.VMEM((BATCH,), jnp.int32),
            pltpu.VMEM((CHUNK_BLKS, 8, CHUNK_COLS), jnp.float32),
            pltpu.VMEM((BATCH,), jnp.int32),
            pltpu.VMEM((BATCH,), jnp.int32),
            pltpu.VMEM((1, SBATCH, 128), jnp.float32),
            pltpu.VMEM((SBATCH,), jnp.int32),
            pltpu.SemaphoreType.DMA,
            pltpu.SemaphoreType.DMA,
        ],
    )
    def k(xr_hbm, xe_hbm, tab_hbm, scr_hbm,
          xr_v, xe_v, wl_ii, wl_b, cbuf, h_ii, h_b, sstage, bstage,
          sem_in, sem_out):
        wid = lax.axis_index("s") * NUM_CORES + lax.axis_index("c")
        nblk_w = BLK_PER_W + jnp.where(wid == NW - 1, NBLK - BLK_PER_W * NW,
                                       0)
        cstart = wid * BLK_PER_W * 128
        cend = cstart + nblk_w * 128
        lanes = lax.iota(jnp.int32, LANES)
        zeros = jnp.zeros((LANES,), jnp.int32)

        def xchunk_body(g, wlfill):
            pltpu.sync_copy(xr_hbm.at[pl.ds(g * XCHUNK, XCHUNK)], xr_v)
            pltpu.sync_copy(xe_hbm.at[pl.ds(g * XCHUNK, XCHUNK)], xe_v)

            def vbody(j, fill):
                sl = pl.ds(j * LANES, LANES)
                ii = xr_v[sl] * ETH_CARD + xe_v[sl]
                m = jnp.logical_and(ii >= cstart, ii < cend)
                mi = m.astype(jnp.int32)
                pos = fill + jnp.cumsum(mi) - 1
                bpos = g * XCHUNK + j * LANES + lanes
                plsc.store_scatter(wl_ii, [pos], ii, mask=m)
                plsc.store_scatter(wl_b, [pos], bpos, mask=m)
                return fill + jnp.sum(mi)

            return lax.fori_loop(0, XCHUNK // LANES, vbody, wlfill)

        wlfill = lax.fori_loop(0, BATCH // XCHUNK, xchunk_body, 0)

        for kk in range(SBATCH // LANES):
            bstage[pl.ds(kk * LANES, LANES)] = BATCH + kk * LANES + lanes

        def flush():
            pltpu.async_copy(sstage.at[0], scr_hbm.at[bstage],
                             sem_out).wait()

        def extract(k16, sfill, hfill, c0):
            sl = pl.ds(k16 * LANES, LANES)
            hv = h_ii[sl]
            bv = h_b[sl]
            nrem = hfill - k16 * LANES
            vm = lanes < nrem
            col = hv - c0
            spos = sfill + lanes
            for f in range(D):
                g = plsc.load_gather(
                    cbuf, [jnp.full((LANES,), f // 8, jnp.int32),
                           jnp.full((LANES,), f % 8, jnp.int32), col],
                    mask=vm)
                plsc.store_scatter(
                    sstage, [zeros, spos,
                             jnp.full((LANES,), f, jnp.int32)], g, mask=vm)
            plsc.store_scatter(bstage, [spos], bv, mask=vm)
            sfill = sfill + jnp.minimum(nrem, LANES)
            do_flush = sfill > SBATCH - LANES
            pl.when(do_flush)(flush)
            return jnp.where(do_flush, 0, sfill)

        def chunk_body(t, sfill):
            c0 = pl.multiple_of(cstart + t * CHUNK_COLS, 128)
            copies = [
                pltpu.async_copy(
                    tab_hbm.at[pl.ds(8 * s, 8), pl.ds(c0, CHUNK_COLS)],
                    cbuf.at[s], sem_in)
                for s in range(CHUNK_BLKS)
            ]
            for cp in copies:
                cp.wait()

            nv = lax.div(wlfill + LANES - 1, LANES)

            def mbody(j, hfill):
                sl = pl.ds(j * LANES, LANES)
                ii = wl_ii[sl]
                bv = wl_b[sl]
                vm = lanes < (wlfill - j * LANES)
                m = jnp.logical_and(
                    vm, jnp.logical_and(ii >= c0, ii < c0 + CHUNK_COLS))
                mi = m.astype(jnp.int32)
                pos = hfill + jnp.cumsum(mi) - 1
                plsc.store_scatter(h_ii, [pos], ii, mask=m)
                plsc.store_scatter(h_b, [pos], bv, mask=m)
                return hfill + jnp.sum(mi)

            hfill = lax.fori_loop(0, nv, mbody, 0)

            nh = lax.div(hfill + LANES - 1, LANES)
            sfill = lax.fori_loop(
                0, nh, lambda k16, s: extract(k16, s, hfill, c0), sfill)
            return sfill

        nchunks = lax.div(nblk_w, CHUNK_BLKS)
        sfill = lax.fori_loop(0, nchunks, chunk_body, 0)
        flush()

    return k


def _make_assemble_kernel():
    mesh = plsc.VectorSubcoreMesh(core_axis_name="c", subcore_axis_name="s")

    @functools.partial(
        pl.kernel,
        mesh=mesh,
        compiler_params=pltpu.CompilerParams(needs_layout_passes=False),
        out_type=jax.ShapeDtypeStruct((3 * D * BATCH,), jnp.float32),
        scratch_types=[
            pltpu.VMEM((D * RACE_CARD,), jnp.float32),
            pltpu.VMEM((D * ETH_CARD,), jnp.float32),
            pltpu.VMEM(((NTAB - TAIL0) * D,), jnp.float32),
            pltpu.VMEM((B_PER_W,), jnp.int32),
            pltpu.VMEM((B_PER_W,), jnp.int32),
            pltpu.VMEM((1, SCHUNK, 128), jnp.float32),
            pltpu.VMEM((3 * D * B_PER_W,), jnp.float32),
            pltpu.SemaphoreType.DMA,
            pltpu.SemaphoreType.DMA,
        ],
    )
    def k(xr_hbm, xe_hbm, racef_hbm, ethf_hbm, tailf_hbm, scr_hbm, out_hbm,
          race_v, eth_v, tail_v, r_idx, e_idx, sbuf, out_v, sem_in,
          sem_out):
        wid = lax.axis_index("s") * NUM_CORES + lax.axis_index("c")
        base = wid * B_PER_W
        lanes = lax.iota(jnp.int32, LANES)
        zeros = jnp.zeros((LANES,), jnp.int32)

        ct_r = pltpu.async_copy(racef_hbm, race_v, sem_in)
        ct_e = pltpu.async_copy(ethf_hbm, eth_v, sem_in)
        ct_t = pltpu.async_copy(tailf_hbm, tail_v, sem_in)
        pltpu.sync_copy(xr_hbm.at[pl.ds(base, B_PER_W)], r_idx)
        pltpu.sync_copy(xe_hbm.at[pl.ds(base, B_PER_W)], e_idx)
        ct_t.wait()

        def schunk_body(t, carry):
            pltpu.async_copy(
                scr_hbm.at[pl.ds(base + t * SCHUNK, SCHUNK)], sbuf.at[0],
                sem_out).wait()

            def blk_body(b, carry2):
                g0 = t * SCHUNK + b * LANES
                sl = pl.ds(g0, LANES)
                ii = r_idx[sl] * ETH_CARD + e_idx[sl]
                tmask = ii >= TAIL0
                taddr = jnp.maximum(ii - TAIL0, 0) * D
                rows = lanes + b * LANES
                for c in range(D):
                    v = plsc.load_gather(
                        sbuf, [zeros, rows,
                               jnp.full((LANES,), c, jnp.int32)])
                    tv = plsc.load_gather(tail_v, [taddr + c])
                    v = jnp.where(tmask, tv, v)
                    out_v[pl.ds((2 * D + c) * B_PER_W + g0, LANES)] = v
                return carry2

            lax.fori_loop(0, SCHUNK // LANES, blk_body, 0)
            return carry

        lax.fori_loop(0, B_PER_W // SCHUNK, schunk_body, 0)

        ct_r.wait()
        ct_e.wait()

        def small_body(j, carry):
            sl = pl.ds(j * LANES, LANES)
            r_vec = r_idx[sl]
            e_vec = e_idx[sl]
            for f in range(D):
                out_v[pl.ds(f * B_PER_W + j * LANES, LANES)] = (
                    plsc.load_gather(race_v, [r_vec * D + f]))
                out_v[pl.ds((D + f) * B_PER_W + j * LANES, LANES)] = (
                    plsc.load_gather(eth_v, [e_vec * D + f]))
            return carry

        lax.fori_loop(0, B_PER_W // LANES, small_body, 0)

        copies = []
        for f in range(3 * D):
            copies.append(pltpu.async_copy(
                out_v.at[pl.ds(f * B_PER_W, B_PER_W)],
                out_hbm.at[pl.ds(f * BATCH + base, B_PER_W)], sem_out))
        for c in copies:
            c.wait()

    return k


_scan_kernel = _make_scan_kernel()
_assemble_kernel = _make_assemble_kernel()


@jax.jit
def kernel(x, race_table, ethnicity_table, interaction_table):
    x = x.astype(jnp.int32)
    xr = x[:, 0]
    xe = x[:, 1]
    racef = race_table.reshape(-1)
    ethf = ethnicity_table.reshape(-1)
    tailf = interaction_table[TAIL0:].reshape(-1)
    tab_t = interaction_table.T
    scratch = _scan_kernel(xr, xe, tab_t)
    out = _assemble_kernel(xr, xe, racef, ethf, tailf, scratch)
    return out.reshape(3 * D, BATCH).T


# speedup vs baseline: 24.7560x; 1.2985x over previous
import functools

import jax
import jax.numpy as jnp
from jax import lax
from jax.experimental import pallas as pl
from jax.experimental.pallas import tpu as pltpu
from jax.experimental.pallas import tpu_sc as plsc

RACE_CARD = 1000
ETH_CARD = 1000
D = 32
BATCH = 16384
NTAB = RACE_CARD * ETH_CARD

NUM_CORES = 2
NUM_SUBCORES = 16
LANES = 16
NW = NUM_CORES * NUM_SUBCORES
B_PER_W = BATCH // NW

NBLK = NTAB // 128
BLK_PER_W = NBLK // NW
CHUNK_BLKS = 4
CHUNK_COLS = CHUNK_BLKS * 128
TAIL0 = NBLK * 128
NDUMP = 64
SCRATCH_ROWS = BATCH + NDUMP
XCHUNK = 2048
SBATCH = 64
SCHUNK = 32


def _make_scan_kernel():
    mesh = plsc.VectorSubcoreMesh(core_axis_name="c", subcore_axis_name="s")

    @functools.partial(
        pl.kernel,
        mesh=mesh,
        compiler_params=pltpu.CompilerParams(needs_layout_passes=False),
        out_type=jax.ShapeDtypeStruct((SCRATCH_ROWS, 128), jnp.float32),
        scratch_types=[
            pltpu.VMEM((XCHUNK,), jnp.int32),
            pltpu.VMEM((XCHUNK,), jnp.int32),
            pltpu.VMEM((BATCH,), jnp.int32),
            pltpu.VMEM((BATCH,), jnp.int32),
            pltpu.VMEM((2, D // 8, 8, CHUNK_COLS), jnp.float32),
            pltpu.VMEM((BATCH,), jnp.int32),
            pltpu.VMEM((BATCH,), jnp.int32),
            pltpu.VMEM((1, SBATCH, 128), jnp.float32),
            pltpu.VMEM((SBATCH,), jnp.int32),
            pltpu.SemaphoreType.DMA,
            pltpu.SemaphoreType.DMA,
        ],
    )
    def k(xr_hbm, xe_hbm, tab_hbm, scr_hbm,
          xr_v, xe_v, wl_ii, wl_b, cbuf, h_ii, h_b, sstage, bstage,
          sem_in, sem_out):
        wid = lax.axis_index("s") * NUM_CORES + lax.axis_index("c")
        nblk_w = BLK_PER_W + jnp.where(wid == NW - 1, NBLK - BLK_PER_W * NW,
                                       0)
        cstart = wid * BLK_PER_W * 128
        cend = cstart + nblk_w * 128
        lanes = lax.iota(jnp.int32, LANES)
        zeros = jnp.zeros((LANES,), jnp.int32)

        def xchunk_body(g, wlfill):
            pltpu.sync_copy(xr_hbm.at[pl.ds(g * XCHUNK, XCHUNK)], xr_v)
            pltpu.sync_copy(xe_hbm.at[pl.ds(g * XCHUNK, XCHUNK)], xe_v)

            def vbody(j, fill):
                sl = pl.ds(j * LANES, LANES)
                ii = xr_v[sl] * ETH_CARD + xe_v[sl]
                m = jnp.logical_and(ii >= cstart, ii < cend)
                mi = m.astype(jnp.int32)
                pos = fill + jnp.cumsum(mi) - 1
                bpos = g * XCHUNK + j * LANES + lanes
                plsc.store_scatter(wl_ii, [pos], ii, mask=m)
                plsc.store_scatter(wl_b, [pos], bpos, mask=m)
                return fill + jnp.sum(mi)

            return lax.fori_loop(0, XCHUNK // LANES, vbody, wlfill)

        wlfill = lax.fori_loop(0, BATCH // XCHUNK, xchunk_body, 0)

        for kk in range(SBATCH // LANES):
            bstage[pl.ds(kk * LANES, LANES)] = BATCH + kk * LANES + lanes

        def flush():
            pltpu.async_copy(sstage.at[0], scr_hbm.at[bstage],
                             sem_out).wait()

        def extract(k16, sfill, hfill, c0, par):
            sl = pl.ds(k16 * LANES, LANES)
            hv = h_ii[sl]
            bv = h_b[sl]
            nrem = hfill - k16 * LANES
            vm = lanes < nrem
            col = hv - c0
            spos = sfill + lanes
            parv = jnp.full((LANES,), 0, jnp.int32) + par
            for f in range(D):
                g = plsc.load_gather(
                    cbuf, [parv, jnp.full((LANES,), f // 8, jnp.int32),
                           jnp.full((LANES,), f % 8, jnp.int32), col],
                    mask=vm)
                plsc.store_scatter(
                    sstage, [zeros, spos,
                             jnp.full((LANES,), f, jnp.int32)], g, mask=vm)
            plsc.store_scatter(bstage, [spos], bv, mask=vm)
            sfill = sfill + jnp.minimum(nrem, LANES)
            do_flush = sfill > SBATCH - LANES
            pl.when(do_flush)(flush)
            return jnp.where(do_flush, 0, sfill)

        nchunks = lax.div(nblk_w, CHUNK_BLKS)

        def start_fetch(t, par):
            c0 = pl.multiple_of(cstart + t * CHUNK_COLS, 128)
            return [
                pltpu.async_copy(
                    tab_hbm.at[pl.ds(8 * s, 8), pl.ds(c0, CHUNK_COLS)],
                    cbuf.at[par, s], sem_in)
                for s in range(D // 8)
            ]

        start_fetch(0, 0)

        def chunk_body(t, sfill):
            par = lax.rem(t, 2)
            for s in range(D // 8):
                pltpu.make_async_copy(
                    tab_hbm.at[pl.ds(8 * s, 8), pl.ds(0, CHUNK_COLS)],
                    cbuf.at[par, s], sem_in).wait()
            pl.when(t + 1 < nchunks)(
                lambda: (start_fetch(t + 1, 1 - par), None)[1])

            c0 = pl.multiple_of(cstart + t * CHUNK_COLS, 128)
            nv = lax.div(wlfill + LANES - 1, LANES)

            def mbody(j, hfill):
                sl = pl.ds(j * LANES, LANES)
                ii = wl_ii[sl]
                bv = wl_b[sl]
                vm = lanes < (wlfill - j * LANES)
                m = jnp.logical_and(
                    vm, jnp.logical_and(ii >= c0, ii < c0 + CHUNK_COLS))
                mi = m.astype(jnp.int32)
                pos = hfill + jnp.cumsum(mi) - 1
                plsc.store_scatter(h_ii, [pos], ii, mask=m)
                plsc.store_scatter(h_b, [pos], bv, mask=m)
                return hfill + jnp.sum(mi)

            hfill = lax.fori_loop(0, nv, mbody, 0)

            nh = lax.div(hfill + LANES - 1, LANES)
            sfill = lax.fori_loop(
                0, nh, lambda k16, s: extract(k16, s, hfill, c0, par),
                sfill)
            return sfill

        sfill = lax.fori_loop(0, nchunks, chunk_body, 0)
        flush()

    return k


def _make_assemble_kernel():
    mesh = plsc.VectorSubcoreMesh(core_axis_name="c", subcore_axis_name="s")

    @functools.partial(
        pl.kernel,
        mesh=mesh,
        compiler_params=pltpu.CompilerParams(needs_layout_passes=False),
        out_type=jax.ShapeDtypeStruct((3 * D, BATCH), jnp.float32),
        scratch_types=[
            pltpu.VMEM((D * RACE_CARD,), jnp.float32),
            pltpu.VMEM((D * ETH_CARD,), jnp.float32),
            pltpu.VMEM(((NTAB - TAIL0) * D,), jnp.float32),
            pltpu.VMEM((B_PER_W,), jnp.int32),
            pltpu.VMEM((B_PER_W,), jnp.int32),
            pltpu.VMEM((2, SCHUNK, 128), jnp.float32),
            pltpu.VMEM((3 * D, B_PER_W), jnp.float32),
            pltpu.SemaphoreType.DMA,
            pltpu.SemaphoreType.DMA,
        ],
    )
    def k(xr_hbm, xe_hbm, racef_hbm, ethf_hbm, tailf_hbm, scr_hbm, out_hbm,
          race_v, eth_v, tail_v, r_idx, e_idx, sbuf, out_v, sem_in,
          sem_out):
        wid = lax.axis_index("s") * NUM_CORES + lax.axis_index("c")
        base = wid * B_PER_W
        lanes = lax.iota(jnp.int32, LANES)

        ct_r = pltpu.async_copy(racef_hbm, race_v, sem_in)
        ct_e = pltpu.async_copy(ethf_hbm, eth_v, sem_in)
        ct_t = pltpu.async_copy(tailf_hbm, tail_v, sem_in)
        pltpu.sync_copy(xr_hbm.at[pl.ds(base, B_PER_W)], r_idx)
        pltpu.sync_copy(xe_hbm.at[pl.ds(base, B_PER_W)], e_idx)

        def sfetch(t, par):
            return pltpu.async_copy(
                scr_hbm.at[pl.ds(base + t * SCHUNK, SCHUNK)], sbuf.at[par],
                sem_out)

        sfetch(0, 0)

        ct_r.wait()
        ct_e.wait()

        def small_body(j, carry):
            sl = pl.ds(j * LANES, LANES)
            r_vec = r_idx[sl]
            e_vec = e_idx[sl]
            for f in range(D):
                out_v[f, pl.ds(j * LANES, LANES)] = (
                    plsc.load_gather(race_v, [r_vec * D + f]))
                out_v[D + f, pl.ds(j * LANES, LANES)] = (
                    plsc.load_gather(eth_v, [e_vec * D + f]))
            return carry

        lax.fori_loop(0, B_PER_W // LANES, small_body, 0)
        ct_t.wait()

        nsc = B_PER_W // SCHUNK

        def schunk_body(t, carry):
            par = lax.rem(t, 2)
            pltpu.make_async_copy(
                scr_hbm.at[pl.ds(base, SCHUNK)], sbuf.at[par],
                sem_out).wait()
            pl.when(t + 1 < nsc)(
                lambda: (sfetch(t + 1, 1 - par), None)[1])
            parv = jnp.full((LANES,), 0, jnp.int32) + par

            def blk_body(b, carry2):
                g0 = t * SCHUNK + b * LANES
                sl = pl.ds(g0, LANES)
                ii = r_idx[sl] * ETH_CARD + e_idx[sl]
                tmask = ii >= TAIL0
                taddr = jnp.maximum(ii - TAIL0, 0) * D
                rows = lanes + b * LANES
                for c in range(D):
                    v = plsc.load_gather(
                        sbuf, [parv, rows,
                               jnp.full((LANES,), c, jnp.int32)])
                    tv = plsc.load_gather(tail_v, [taddr + c])
                    v = jnp.where(tmask, tv, v)
                    out_v[2 * D + c, pl.ds(g0, LANES)] = v
                return carry2

            lax.fori_loop(0, SCHUNK // LANES, blk_body, 0)
            return carry

        lax.fori_loop(0, nsc, schunk_body, 0)

        pltpu.sync_copy(out_v, out_hbm.at[:, pl.ds(base, B_PER_W)])

    return k


_scan_kernel = _make_scan_kernel()
_assemble_kernel = _make_assemble_kernel()


@jax.jit
def kernel(x, race_table, ethnicity_table, interaction_table):
    x = x.astype(jnp.int32)
    xr = x[:, 0]
    xe = x[:, 1]
    racef = race_table.reshape(-1)
    ethf = ethnicity_table.reshape(-1)
    tailf = interaction_table[TAIL0:].reshape(-1)
    tab_t = interaction_table.T
    scratch = _scan_kernel(xr, xe, tab_t)
    out = _assemble_kernel(xr, xe, racef, ethf, tailf, scratch)
    return out.T


# speedup vs baseline: 26.9020x; 1.0867x over previous
import functools

import jax
import jax.numpy as jnp
from jax import lax
from jax.experimental import pallas as pl
from jax.experimental.pallas import tpu as pltpu
from jax.experimental.pallas import tpu_sc as plsc

RACE_CARD = 1000
ETH_CARD = 1000
D = 32
BATCH = 16384
NTAB = RACE_CARD * ETH_CARD

NUM_CORES = 2
NUM_SUBCORES = 16
LANES = 16
NW = NUM_CORES * NUM_SUBCORES
B_PER_W = BATCH // NW

NBLK = NTAB // 128
BLK_PER_W = NBLK // NW
CHUNK_BLKS = 4
CHUNK_COLS = CHUNK_BLKS * 128
TAIL0 = NBLK * 128
NDUMP = 64
SCRATCH_ROWS = BATCH + NDUMP
XCHUNK = 2048
SBATCH = 64
SCHUNK = 32


def _make_scan_kernel():
    mesh = plsc.VectorSubcoreMesh(core_axis_name="c", subcore_axis_name="s")

    @functools.partial(
        pl.kernel,
        mesh=mesh,
        compiler_params=pltpu.CompilerParams(needs_layout_passes=False),
        out_type=jax.ShapeDtypeStruct((SCRATCH_ROWS, 128), jnp.float32),
        scratch_types=[
            pltpu.VMEM((XCHUNK,), jnp.int32),
            pltpu.VMEM((XCHUNK,), jnp.int32),
            pltpu.VMEM((BATCH,), jnp.int32),
            pltpu.VMEM((BATCH,), jnp.int32),
            pltpu.VMEM((3, D // 8, 8, CHUNK_COLS), jnp.float32),
            pltpu.VMEM((BATCH,), jnp.int32),
            pltpu.VMEM((BATCH,), jnp.int32),
            pltpu.VMEM((1, SBATCH, 128), jnp.float32),
            pltpu.VMEM((SBATCH,), jnp.int32),
            pltpu.SemaphoreType.DMA,
            pltpu.SemaphoreType.DMA,
        ],
    )
    def k(xr_hbm, xe_hbm, tab_hbm, scr_hbm,
          xr_v, xe_v, wl_ii, wl_b, cbuf, h_ii, h_b, sstage, bstage,
          sem_in, sem_out):
        wid = lax.axis_index("s") * NUM_CORES + lax.axis_index("c")
        nblk_w = BLK_PER_W + jnp.where(wid == NW - 1, NBLK - BLK_PER_W * NW,
                                       0)
        cstart = wid * BLK_PER_W * 128
        cend = cstart + nblk_w * 128
        lanes = lax.iota(jnp.int32, LANES)
        zeros = jnp.zeros((LANES,), jnp.int32)

        def xchunk_body(g, wlfill):
            pltpu.sync_copy(xr_hbm.at[pl.ds(g * XCHUNK, XCHUNK)], xr_v)
            pltpu.sync_copy(xe_hbm.at[pl.ds(g * XCHUNK, XCHUNK)], xe_v)

            def vbody(j, fill):
                sl = pl.ds(j * LANES, LANES)
                ii = xr_v[sl] * ETH_CARD + xe_v[sl]
                m = jnp.logical_and(ii >= cstart, ii < cend)
                mi = m.astype(jnp.int32)
                pos = fill + jnp.cumsum(mi) - 1
                bpos = g * XCHUNK + j * LANES + lanes
                plsc.store_scatter(wl_ii, [pos], ii, mask=m)
                plsc.store_scatter(wl_b, [pos], bpos, mask=m)
                return fill + jnp.sum(mi)

            return lax.fori_loop(0, XCHUNK // LANES, vbody, wlfill)

        wlfill = lax.fori_loop(0, BATCH // XCHUNK, xchunk_body, 0)

        for kk in range(SBATCH // LANES):
            bstage[pl.ds(kk * LANES, LANES)] = BATCH + kk * LANES + lanes

        def flush():
            pltpu.async_copy(sstage.at[0], scr_hbm.at[bstage],
                             sem_out).wait()

        def extract(k16, sfill, hfill, c0, par):
            sl = pl.ds(k16 * LANES, LANES)
            hv = h_ii[sl]
            bv = h_b[sl]
            nrem = hfill - k16 * LANES
            vm = lanes < nrem
            col = hv - c0
            spos = sfill + lanes
            parv = jnp.full((LANES,), 0, jnp.int32) + par
            for f in range(D):
                g = plsc.load_gather(
                    cbuf, [parv, jnp.full((LANES,), f // 8, jnp.int32),
                           jnp.full((LANES,), f % 8, jnp.int32), col],
                    mask=vm)
                plsc.store_scatter(
                    sstage, [zeros, spos,
                             jnp.full((LANES,), f, jnp.int32)], g, mask=vm)
            plsc.store_scatter(bstage, [spos], bv, mask=vm)
            sfill = sfill + jnp.minimum(nrem, LANES)
            do_flush = sfill > SBATCH - LANES
            pl.when(do_flush)(flush)
            return jnp.where(do_flush, 0, sfill)

        nchunks = lax.div(nblk_w, CHUNK_BLKS)

        def start_fetch(t, par):
            c0 = pl.multiple_of(cstart + t * CHUNK_COLS, 128)
            return [
                pltpu.async_copy(
                    tab_hbm.at[pl.ds(8 * s, 8), pl.ds(c0, CHUNK_COLS)],
                    cbuf.at[par, s], sem_in)
                for s in range(D // 8)
            ]

        start_fetch(0, 0)
        pl.when(1 < nchunks)(lambda: (start_fetch(1, 1), None)[1])

        def chunk_body(t, sfill):
            par = lax.rem(t, 3)
            for s in range(D // 8):
                pltpu.make_async_copy(
                    tab_hbm.at[pl.ds(8 * s, 8), pl.ds(0, CHUNK_COLS)],
                    cbuf.at[par, s], sem_in).wait()
            pl.when(t + 2 < nchunks)(
                lambda: (start_fetch(t + 2, lax.rem(t + 2, 3)), None)[1])

            c0 = pl.multiple_of(cstart + t * CHUNK_COLS, 128)
            nv = lax.div(wlfill + LANES - 1, LANES)

            def mbody(j, hfill):
                sl = pl.ds(j * LANES, LANES)
                ii = wl_ii[sl]
                bv = wl_b[sl]
                vm = lanes < (wlfill - j * LANES)
                m = jnp.logical_and(
                    vm, jnp.logical_and(ii >= c0, ii < c0 + CHUNK_COLS))
                mi = m.astype(jnp.int32)
                pos = hfill + jnp.cumsum(mi) - 1
                plsc.store_scatter(h_ii, [pos], ii, mask=m)
                plsc.store_scatter(h_b, [pos], bv, mask=m)
                return hfill + jnp.sum(mi)

            hfill = lax.fori_loop(0, nv, mbody, 0)

            nh = lax.div(hfill + LANES - 1, LANES)
            sfill = lax.fori_loop(
                0, nh, lambda k16, s: extract(k16, s, hfill, c0, par),
                sfill)
            return sfill

        sfill = lax.fori_loop(0, nchunks, chunk_body, 0)
        flush()

    return k


def _make_assemble_kernel():
    mesh = plsc.VectorSubcoreMesh(core_axis_name="c", subcore_axis_name="s")

    @functools.partial(
        pl.kernel,
        mesh=mesh,
        compiler_params=pltpu.CompilerParams(needs_layout_passes=False),
        out_type=jax.ShapeDtypeStruct((3 * D, BATCH), jnp.float32),
        scratch_types=[
            pltpu.VMEM((D * RACE_CARD,), jnp.float32),
            pltpu.VMEM((D * ETH_CARD,), jnp.float32),
            pltpu.VMEM(((NTAB - TAIL0) * D,), jnp.float32),
            pltpu.VMEM((B_PER_W,), jnp.int32),
            pltpu.VMEM((B_PER_W,), jnp.int32),
            pltpu.VMEM((2, SCHUNK, 128), jnp.float32),
            pltpu.VMEM((3 * D, B_PER_W), jnp.float32),
            pltpu.SemaphoreType.DMA,
            pltpu.SemaphoreType.DMA,
        ],
    )
    def k(xr_hbm, xe_hbm, racef_hbm, ethf_hbm, tailf_hbm, scr_hbm, out_hbm,
          race_v, eth_v, tail_v, r_idx, e_idx, sbuf, out_v, sem_in,
          sem_out):
        wid = lax.axis_index("s") * NUM_CORES + lax.axis_index("c")
        base = wid * B_PER_W
        lanes = lax.iota(jnp.int32, LANES)

        ct_r = pltpu.async_copy(racef_hbm, race_v, sem_in)
        ct_e = pltpu.async_copy(ethf_hbm, eth_v, sem_in)
        ct_t = pltpu.async_copy(tailf_hbm, tail_v, sem_in)
        pltpu.sync_copy(xr_hbm.at[pl.ds(base, B_PER_W)], r_idx)
        pltpu.sync_copy(xe_hbm.at[pl.ds(base, B_PER_W)], e_idx)

        def sfetch(t, par):
            return pltpu.async_copy(
                scr_hbm.at[pl.ds(base + t * SCHUNK, SCHUNK)], sbuf.at[par],
                sem_out)

        sfetch(0, 0)

        ct_r.wait()
        ct_e.wait()

        def small_body(j, carry):
            sl = pl.ds(j * LANES, LANES)
            r_vec = r_idx[sl]
            e_vec = e_idx[sl]
            for f in range(D):
                out_v[f, pl.ds(j * LANES, LANES)] = (
                    plsc.load_gather(race_v, [r_vec * D + f]))
                out_v[D + f, pl.ds(j * LANES, LANES)] = (
                    plsc.load_gather(eth_v, [e_vec * D + f]))
            return carry

        lax.fori_loop(0, B_PER_W // LANES, small_body, 0)
        ct_t.wait()

        nsc = B_PER_W // SCHUNK

        def schunk_body(t, carry):
            par = lax.rem(t, 2)
            pltpu.make_async_copy(
                scr_hbm.at[pl.ds(base, SCHUNK)], sbuf.at[par],
                sem_out).wait()
            pl.when(t + 1 < nsc)(
                lambda: (sfetch(t + 1, 1 - par), None)[1])
            parv = jnp.full((LANES,), 0, jnp.int32) + par

            def blk_body(b, carry2):
                g0 = t * SCHUNK + b * LANES
                sl = pl.ds(g0, LANES)
                ii = r_idx[sl] * ETH_CARD + e_idx[sl]
                tmask = ii >= TAIL0
                taddr = jnp.maximum(ii - TAIL0, 0) * D
                rows = lanes + b * LANES
                for c in range(D):
                    v = plsc.load_gather(
                        sbuf, [parv, rows,
                               jnp.full((LANES,), c, jnp.int32)])
                    tv = plsc.load_gather(tail_v, [taddr + c])
                    v = jnp.where(tmask, tv, v)
                    out_v[2 * D + c, pl.ds(g0, LANES)] = v
                return carry2

            lax.fori_loop(0, SCHUNK // LANES, blk_body, 0)
            return carry

        lax.fori_loop(0, nsc, schunk_body, 0)

        pltpu.sync_copy(out_v, out_hbm.at[:, pl.ds(base, B_PER_W)])

    return k


_scan_kernel = _make_scan_kernel()
_assemble_kernel = _make_assemble_kernel()


@jax.jit
def kernel(x, race_table, ethnicity_table, interaction_table):
    x = x.astype(jnp.int32)
    xr = x[:, 0]
    xe = x[:, 1]
    racef = race_table.reshape(-1)
    ethf = ethnicity_table.reshape(-1)
    tailf = interaction_table[TAIL0:].reshape(-1)
    tab_t = interaction_table.T
    scratch = _scan_kernel(xr, xe, tab_t)
    out = _assemble_kernel(xr, xe, racef, ethf, tailf, scratch)
    return out.T


# speedup vs baseline: 28.0781x; 1.0437x over previous
import functools

import jax
import jax.numpy as jnp
from jax import lax
from jax.experimental.layout import Layout, with_layout_constraint
from jax.experimental import pallas as pl
from jax.experimental.pallas import tpu as pltpu
from jax.experimental.pallas import tpu_sc as plsc

RACE_CARD = 1000
ETH_CARD = 1000
D = 32
BATCH = 16384
NTAB = RACE_CARD * ETH_CARD

NUM_CORES = 2
NUM_SUBCORES = 16
LANES = 16
NW = NUM_CORES * NUM_SUBCORES
B_PER_W = BATCH // NW

NBLK = NTAB // 128
BLK_PER_W = NBLK // NW
CHUNK_BLKS = 4
CHUNK_COLS = CHUNK_BLKS * 128
TAIL0 = NBLK * 128
NDUMP = 64
SCRATCH_ROWS = BATCH + NDUMP
XCHUNK = 1024
SBATCH = 64
SCHUNK = 32


def _make_scan_kernel():
    mesh = plsc.VectorSubcoreMesh(core_axis_name="c", subcore_axis_name="s")

    @functools.partial(
        pl.kernel,
        mesh=mesh,
        compiler_params=pltpu.CompilerParams(needs_layout_passes=False),
        out_type=jax.ShapeDtypeStruct((SCRATCH_ROWS, 128), jnp.float32),
        scratch_types=[
            pltpu.VMEM((2 * XCHUNK,), jnp.int32),
            pltpu.VMEM((2 * XCHUNK,), jnp.int32),
            pltpu.VMEM((BATCH,), jnp.int32),
            pltpu.VMEM((BATCH,), jnp.int32),
            pltpu.VMEM((3, D // 8, 8, CHUNK_COLS), jnp.float32),
            pltpu.VMEM((BATCH,), jnp.int32),
            pltpu.VMEM((BATCH,), jnp.int32),
            pltpu.VMEM((1, SBATCH, 128), jnp.float32),
            pltpu.VMEM((SBATCH,), jnp.int32),
            pltpu.SemaphoreType.DMA,
            pltpu.SemaphoreType.DMA,
        ],
    )
    def k(xr_hbm, xe_hbm, tab_hbm, scr_hbm,
          xr_v, xe_v, wl_ii, wl_b, cbuf, h_ii, h_b, sstage, bstage,
          sem_in, sem_out):
        wid = lax.axis_index("s") * NUM_CORES + lax.axis_index("c")
        nblk_w = BLK_PER_W + jnp.where(wid == NW - 1, NBLK - BLK_PER_W * NW,
                                       0)
        cstart = wid * BLK_PER_W * 128
        cend = cstart + nblk_w * 128
        lanes = lax.iota(jnp.int32, LANES)
        zeros = jnp.zeros((LANES,), jnp.int32)

        def xfetch(g):
            return (pltpu.async_copy(
                        xr_hbm.at[pl.ds(g * XCHUNK, XCHUNK)],
                        xr_v.at[pl.ds(lax.rem(g, 2) * XCHUNK, XCHUNK)],
                        sem_in),
                    pltpu.async_copy(
                        xe_hbm.at[pl.ds(g * XCHUNK, XCHUNK)],
                        xe_v.at[pl.ds(lax.rem(g, 2) * XCHUNK, XCHUNK)],
                        sem_in))

        xfetch(0)
        xfetch(1)

        def xchunk_body(g, wlfill):
            xoff = lax.rem(g, 2) * XCHUNK
            for _ in range(2):
                pltpu.make_async_copy(
                    xr_hbm.at[pl.ds(0, XCHUNK)],
                    xr_v.at[pl.ds(0, XCHUNK)], sem_in).wait()
            pl.when(g + 2 < BATCH // XCHUNK)(
                lambda: (xfetch(g + 2), None)[1])

            def vbody(j, fill):
                sl = pl.ds(xoff + j * LANES, LANES)
                ii = xr_v[sl] * ETH_CARD + xe_v[sl]
                m = jnp.logical_and(ii >= cstart, ii < cend)
                mi = m.astype(jnp.int32)
                pos = fill + jnp.cumsum(mi) - 1
                bpos = g * XCHUNK + j * LANES + lanes
                plsc.store_scatter(wl_ii, [pos], ii, mask=m)
                plsc.store_scatter(wl_b, [pos], bpos, mask=m)
                return fill + jnp.sum(mi)

            return lax.fori_loop(0, XCHUNK // LANES, vbody, wlfill)

        wlfill = lax.fori_loop(0, BATCH // XCHUNK, xchunk_body, 0)

        for kk in range(SBATCH // LANES):
            bstage[pl.ds(kk * LANES, LANES)] = BATCH + kk * LANES + lanes

        def flush():
            pltpu.async_copy(sstage.at[0], scr_hbm.at[bstage],
                             sem_out).wait()

        def extract(k16, sfill, hfill, c0, par):
            sl = pl.ds(k16 * LANES, LANES)
            hv = h_ii[sl]
            bv = h_b[sl]
            nrem = hfill - k16 * LANES
            vm = lanes < nrem
            col = hv - c0
            spos = sfill + lanes
            parv = jnp.full((LANES,), 0, jnp.int32) + par
            for f in range(D):
                g = plsc.load_gather(
                    cbuf, [parv, jnp.full((LANES,), f // 8, jnp.int32),
                           jnp.full((LANES,), f % 8, jnp.int32), col],
                    mask=vm)
                plsc.store_scatter(
                    sstage, [zeros, spos,
                             jnp.full((LANES,), f, jnp.int32)], g, mask=vm)
            plsc.store_scatter(bstage, [spos], bv, mask=vm)
            sfill = sfill + jnp.minimum(nrem, LANES)
            do_flush = sfill > SBATCH - LANES
            pl.when(do_flush)(flush)
            return jnp.where(do_flush, 0, sfill)

        nchunks = lax.div(nblk_w, CHUNK_BLKS)

        def start_fetch(t, par):
            c0 = pl.multiple_of(cstart + t * CHUNK_COLS, 128)
            return [
                pltpu.async_copy(
                    tab_hbm.at[pl.ds(8 * s, 8), pl.ds(c0, CHUNK_COLS)],
                    cbuf.at[par, s], sem_in)
                for s in range(D // 8)
            ]

        start_fetch(0, 0)
        pl.when(1 < nchunks)(lambda: (start_fetch(1, 1), None)[1])

        def chunk_body(t, sfill):
            par = lax.rem(t, 3)
            for s in range(D // 8):
                pltpu.make_async_copy(
                    tab_hbm.at[pl.ds(8 * s, 8), pl.ds(0, CHUNK_COLS)],
                    cbuf.at[par, s], sem_in).wait()
            pl.when(t + 2 < nchunks)(
                lambda: (start_fetch(t + 2, lax.rem(t + 2, 3)), None)[1])

            c0 = pl.multiple_of(cstart + t * CHUNK_COLS, 128)
            nv = lax.div(wlfill + LANES - 1, LANES)

            def mbody(j, hfill):
                sl = pl.ds(j * LANES, LANES)
                ii = wl_ii[sl]
                bv = wl_b[sl]
                vm = lanes < (wlfill - j * LANES)
                m = jnp.logical_and(
                    vm, jnp.logical_and(ii >= c0, ii < c0 + CHUNK_COLS))
                mi = m.astype(jnp.int32)
                pos = hfill + jnp.cumsum(mi) - 1
                plsc.store_scatter(h_ii, [pos], ii, mask=m)
                plsc.store_scatter(h_b, [pos], bv, mask=m)
                return hfill + jnp.sum(mi)

            hfill = lax.fori_loop(0, nv, mbody, 0)

            nh = lax.div(hfill + LANES - 1, LANES)
            sfill = lax.fori_loop(
                0, nh, lambda k16, s: extract(k16, s, hfill, c0, par),
                sfill)
            return sfill

        sfill = lax.fori_loop(0, nchunks, chunk_body, 0)
        flush()

    return k


def _make_assemble_kernel():
    mesh = plsc.VectorSubcoreMesh(core_axis_name="c", subcore_axis_name="s")

    @functools.partial(
        pl.kernel,
        mesh=mesh,
        compiler_params=pltpu.CompilerParams(needs_layout_passes=False),
        out_type=jax.ShapeDtypeStruct((3 * D, BATCH), jnp.float32),
        scratch_types=[
            pltpu.VMEM((D * RACE_CARD,), jnp.float32),
            pltpu.VMEM((D * ETH_CARD,), jnp.float32),
            pltpu.VMEM(((NTAB - TAIL0) * D,), jnp.float32),
            pltpu.VMEM((B_PER_W,), jnp.int32),
            pltpu.VMEM((B_PER_W,), jnp.int32),
            pltpu.VMEM((2, SCHUNK, 128), jnp.float32),
            pltpu.VMEM((3 * D, B_PER_W), jnp.float32),
            pltpu.SemaphoreType.DMA,
            pltpu.SemaphoreType.DMA,
        ],
    )
    def k(xr_hbm, xe_hbm, racef_hbm, ethf_hbm, tailf_hbm, scr_hbm, out_hbm,
          race_v, eth_v, tail_v, r_idx, e_idx, sbuf, out_v, sem_in,
          sem_out):
        wid = lax.axis_index("s") * NUM_CORES + lax.axis_index("c")
        base = wid * B_PER_W
        lanes = lax.iota(jnp.int32, LANES)

        ct_r = pltpu.async_copy(racef_hbm, race_v, sem_in)
        ct_e = pltpu.async_copy(ethf_hbm, eth_v, sem_in)
        ct_t = pltpu.async_copy(tailf_hbm, tail_v, sem_in)
        pltpu.sync_copy(xr_hbm.at[pl.ds(base, B_PER_W)], r_idx)
        pltpu.sync_copy(xe_hbm.at[pl.ds(base, B_PER_W)], e_idx)

        def sfetch(t, par):
            return pltpu.async_copy(
                scr_hbm.at[pl.ds(base + t * SCHUNK, SCHUNK)], sbuf.at[par],
                sem_out)

        sfetch(0, 0)
        sfetch(1, 1)

        ct_r.wait()
        ct_e.wait()

        def small_body(j, carry):
            sl = pl.ds(j * LANES, LANES)
            r_vec = r_idx[sl]
            e_vec = e_idx[sl]
            for f in range(D):
                out_v[f, pl.ds(j * LANES, LANES)] = (
                    plsc.load_gather(race_v, [r_vec * D + f]))
                out_v[D + f, pl.ds(j * LANES, LANES)] = (
                    plsc.load_gather(eth_v, [e_vec * D + f]))
            return carry

        lax.fori_loop(0, B_PER_W // LANES, small_body, 0)
        ct_t.wait()

        nsc = B_PER_W // SCHUNK

        def schunk_body(t, carry):
            par = lax.rem(t, 2)
            pltpu.make_async_copy(
                scr_hbm.at[pl.ds(base, SCHUNK)], sbuf.at[par],
                sem_out).wait()
            parv = jnp.full((LANES,), 0, jnp.int32) + par

            def blk_body(b, carry2):
                g0 = t * SCHUNK + b * LANES
                sl = pl.ds(g0, LANES)
                ii = r_idx[sl] * ETH_CARD + e_idx[sl]
                tmask = ii >= TAIL0
                taddr = jnp.maximum(ii - TAIL0, 0) * D
                rows = lanes + b * LANES
                for c in range(D):
                    v = plsc.load_gather(
                        sbuf, [parv, rows,
                               jnp.full((LANES,), c, jnp.int32)])
                    tv = plsc.load_gather(tail_v, [taddr + c])
                    v = jnp.where(tmask, tv, v)
                    out_v[2 * D + c, pl.ds(g0, LANES)] = v
                return carry2

            lax.fori_loop(0, SCHUNK // LANES, blk_body, 0)
            pl.when(t + 2 < nsc)(
                lambda: (sfetch(t + 2, par), None)[1])
            return carry

        lax.fori_loop(0, nsc, schunk_body, 0)

        pltpu.sync_copy(out_v, out_hbm.at[:, pl.ds(base, B_PER_W)])

    return k


_scan_kernel = _make_scan_kernel()
_assemble_kernel = _make_assemble_kernel()


@jax.jit
def kernel(x, race_table, ethnicity_table, interaction_table):
    xc = x.astype(jnp.int32).T.reshape(-1)
    xr = xc[:BATCH]
    xe = xc[BATCH:]
    racef = race_table.reshape(-1)
    ethf = ethnicity_table.reshape(-1)
    tailf = interaction_table[TAIL0:].reshape(-1)
    tab_t = interaction_table.T
    scratch = _scan_kernel(xr, xe, tab_t)
    out = _assemble_kernel(xr, xe, racef, ethf, tailf, scratch)
    out_t = out.T
    return with_layout_constraint(out_t, Layout((0, 1)))
